# baseline (speedup 1.0000x reference)
import functools

import jax
import jax.numpy as jnp
from jax import lax
from jax.experimental import pallas as pl
from jax.experimental.pallas import tpu as pltpu
from jax.experimental.pallas import tpu_sc as plsc

_N = 10000
_M = 32
_A = 128
_NBR = 16
_NCONV = 3
_H = 192
_B = 100
_NI = 28
_CU = 29
_E = _N * _M
_NW = 32
_PW = _E // _NW
_CH = 400
_NCH = _PW // _CH

_T = 80
_GRID = _N // _T


def _softplus(x):
    return jnp.maximum(x, 0.0) + jnp.log(1.0 + jnp.exp(-jnp.abs(x)))


def _sigmoid(x):
    return 1.0 / (1.0 + jnp.exp(-x))


def _sc_gather(idx_flat, table):
    mesh = plsc.VectorSubcoreMesh(core_axis_name="c", subcore_axis_name="s")

    @functools.partial(
        pl.kernel,
        out_type=jax.ShapeDtypeStruct((_E, _A), jnp.float32),
        mesh=mesh,
        scratch_types=[
            pltpu.VMEM((_PW,), jnp.int32),
            pltpu.VMEM((_CH, _A), jnp.float32),
            pltpu.VMEM((_CH, _A), jnp.float32),
            pltpu.SemaphoreType.DMA,
            pltpu.SemaphoreType.DMA,
        ],
    )
    def gk(idx_hbm, tab_hbm, out_hbm, idx_v, rows_a, rows_b, g0, g1):
        wid = lax.axis_index("s") * 2 + lax.axis_index("c")
        base = wid * _PW
        pltpu.sync_copy(idx_hbm.at[pl.ds(base, _PW)], idx_v)
        gsem = (g0, g1)
        bufs = (rows_a, rows_b)

        def gather(c, b):
            pltpu.async_copy(
                tab_hbm.at[idx_v.at[pl.ds(c * _CH, _CH)]], bufs[b], gsem[b]
            )

        def gather_wait(c, b):
            pltpu.make_async_copy(
                tab_hbm.at[idx_v.at[pl.ds(c * _CH, _CH)]], bufs[b], gsem[b]
            ).wait()

        gather(0, 0)

        def outer(o, carry):
            c0 = o * 2
            for b in (0, 1):
                c = c0 + b

                @pl.when(c + 1 < _NCH)
                def _():
                    gather(c + 1, 1 - b)

                @pl.when(c < _NCH)
                def _():
                    gather_wait(c, b)
                    pltpu.sync_copy(
                        bufs[b], out_hbm.at[pl.ds(base + c * _CH, _CH)]
                    )

            return carry

        lax.fori_loop(0, (_NCH + 1) // 2, outer, 0)

    return gk(idx_flat, table)


def _embed(atom_fea, emb_W, emb_b):
    tm = 2000

    def body(x_ref, w_ref, b_ref, o_ref):
        o_ref[...] = (
            jnp.dot(x_ref[...], w_ref[...], preferred_element_type=jnp.float32)
            + b_ref[...]
        )

    return pl.pallas_call(
        body,
        grid=(_N // tm,),
        in_specs=[
            pl.BlockSpec((tm, _A), lambda i: (i, 0)),
            pl.BlockSpec((_A, _A), lambda i: (0, 0)),
            pl.BlockSpec((1, _A), lambda i: (0, 0)),
        ],
        out_specs=pl.BlockSpec((tm, _A), lambda i: (i, 0)),
        out_shape=jax.ShapeDtypeStruct((_N, _A), jnp.float32),
    )(atom_fea, emb_W, emb_b.reshape(1, _A))


def _gated_tile(x_ref, g_ref, e_ref, w1_ref, w2_ref, w3_ref, b_ref):
    xw = jnp.dot(x_ref[...], w1_ref[...], preferred_element_type=jnp.float32)
    g2 = g_ref[...].reshape(_T * _M, _A).astype(jnp.bfloat16)
    gw = jnp.dot(g2, w2_ref[...], preferred_element_type=jnp.float32)
    e2 = e_ref[...].reshape(_T * _M, _NBR)
    ew = jnp.dot(e2, w3_ref[...], preferred_element_type=jnp.float32)
    xrep = jnp.broadcast_to(xw[:, None, :], (_T, _M, 2 * _A)).reshape(_T * _M, 2 * _A)
    return gw + ew + b_ref[...] + xrep


def _conv_stats(x, gath3, nbr_fea, w, b):
    def body(x_ref, g_ref, e_ref, w1_ref, w2_ref, w3_ref, b_ref, o_ref):
        gated = _gated_tile(x_ref, g_ref, e_ref, w1_ref, w2_ref, w3_ref, b_ref)
        s = jnp.sum(gated, axis=0).reshape(1, 2 * _A)
        ss = jnp.sum(gated * gated, axis=0).reshape(1, 2 * _A)
        part = jnp.concatenate([s, ss, jnp.zeros((6, 2 * _A), jnp.float32)], axis=0)

        @pl.when(pl.program_id(0) == 0)
        def _():
            o_ref[...] = jnp.zeros_like(o_ref)

        o_ref[...] += part

    return pl.pallas_call(
        body,
        grid=(_GRID,),
        in_specs=[
            pl.BlockSpec((_T, _A), lambda i: (i, 0)),
            pl.BlockSpec((_T, _M, _A), lambda i: (i, 0, 0)),
            pl.BlockSpec((_T, _M, _NBR), lambda i: (i, 0, 0)),
            pl.BlockSpec((_A, 2 * _A), lambda i: (0, 0)),
            pl.BlockSpec((_A, 2 * _A), lambda i: (0, 0)),
            pl.BlockSpec((_NBR, 2 * _A), lambda i: (0, 0)),
            pl.BlockSpec((1, 2 * _A), lambda i: (0, 0)),
        ],
        out_specs=pl.BlockSpec((8, 2 * _A), lambda i: (0, 0)),
        out_shape=jax.ShapeDtypeStruct((8, 2 * _A), jnp.float32),
    )(x, gath3, nbr_fea, w[:_A], w[_A:2 * _A], w[2 * _A:], b.reshape(1, 2 * _A))


def _gram_stats(x_bf, gath3, nbr_bf, w, b, g1v, b1v):
    f32 = jnp.float32
    dn = (((0,), (0,)), ((), ()))

    def body(x_ref, g_ref, e_ref, w1_ref, w2_ref, w3_ref, bb_ref, g1_ref, b1_ref,
             sc_ref, bf_ref, p1_s, p3_s, c22_s, c23_s, p4_s, q_s, vx_s, vg_s, ve_s):
        i = pl.program_id(0)
        xt = x_ref[...]
        g32 = g_ref[...]
        gt = g32.astype(jnp.bfloat16).reshape(_T * _M, _A)
        et = e_ref[...].reshape(_T * _M, _NBR)
        st32 = jnp.sum(g32, axis=1)
        est32 = jnp.sum(e_ref[...].astype(f32), axis=1)
        stb = st32.astype(jnp.bfloat16)
        estb = est32.astype(jnp.bfloat16)

        @pl.when(i == 0)
        def _():
            p1_s[...] = jnp.zeros_like(p1_s)
            p3_s[...] = jnp.zeros_like(p3_s)
            c22_s[...] = jnp.zeros_like(c22_s)
            c23_s[...] = jnp.zeros_like(c23_s)
            p4_s[...] = jnp.zeros_like(p4_s)
            q_s[...] = jnp.zeros_like(q_s)
            vx_s[...] = jnp.zeros_like(vx_s)
            vg_s[...] = jnp.zeros_like(vg_s)
            ve_s[...] = jnp.zeros_like(ve_s)

        p1_s[...] += lax.dot_general(xt, xt, dn, preferred_element_type=f32)
        p3_s[...] += lax.dot_general(xt, stb, dn, preferred_element_type=f32)
        c22_s[...] += lax.dot_general(gt, gt, dn, preferred_element_type=f32)
        c23_s[...] += lax.dot_general(gt, et, dn, preferred_element_type=f32)
        p4_s[...] += lax.dot_general(xt, estb, dn, preferred_element_type=f32)
        q_s[...] += lax.dot_general(et, et, dn, preferred_element_type=f32)
        vx_s[...] += jnp.sum(xt.astype(f32), axis=0).reshape(1, _A)
        vg_s[...] += jnp.sum(st32, axis=0).reshape(1, _A)
        ve_s[...] += jnp.sum(est32, axis=0).reshape(1, _NBR)

        @pl.when(i == _GRID - 1)
        def _():
            w1 = w1_ref[...]
            w2 = w2_ref[...]
            w3 = w3_ref[...]
            bb = bb_ref[...]
            n = f32(_E)
            c11 = p1_s[...] * f32(_M)
            t1 = jnp.sum(w1 * jnp.dot(c11, w1, preferred_element_type=f32), axis=0)
            t2 = jnp.sum(w2 * jnp.dot(c22_s[...], w2, preferred_element_type=f32), axis=0)
            t3 = jnp.sum(w3 * jnp.dot(q_s[...], w3, preferred_element_type=f32), axis=0)
            c1 = jnp.sum(w1 * jnp.dot(p3_s[...], w2, preferred_element_type=f32), axis=0)
            c2 = jnp.sum(w1 * jnp.dot(p4_s[...], w3, preferred_element_type=f32), axis=0)
            c3 = jnp.sum(w2 * jnp.dot(c23_s[...], w3, preferred_element_type=f32), axis=0)
            d = (t1 + t2 + t3 + 2.0 * (c1 + c2 + c3)).reshape(1, 2 * _A)
            sdot = (
                f32(_M) * jnp.dot(vx_s[...], w1, preferred_element_type=f32)
                + jnp.dot(vg_s[...], w2, preferred_element_type=f32)
                + jnp.dot(ve_s[...], w3, preferred_element_type=f32)
                + n * bb
            )
            mu = sdot / n
            sumsq = d + 2.0 * bb * sdot - n * bb * bb
            var = sumsq / n - mu * mu
            sc = g1_ref[...] / jnp.sqrt(var + 1e-5)
            sc_ref[...] = sc
            bf_ref[...] = (bb - mu) * sc + b1_ref[...]

    return pl.pallas_call(
        body,
        grid=(_GRID,),
        in_specs=[
            pl.BlockSpec((_T, _A), lambda i: (i, 0)),
            pl.BlockSpec((_T, _M, _A), lambda i: (i, 0, 0)),
            pl.BlockSpec((_T, _M, _NBR), lambda i: (i, 0, 0)),
            pl.BlockSpec((_A, 2 * _A), lambda i: (0, 0)),
            pl.BlockSpec((_A, 2 * _A), lambda i: (0, 0)),
            pl.BlockSpec((_NBR, 2 * _A), lambda i: (0, 0)),
            pl.BlockSpec((1, 2 * _A), lambda i: (0, 0)),
            pl.BlockSpec((1, 2 * _A), lambda i: (0, 0)),
            pl.BlockSpec((1, 2 * _A), lambda i: (0, 0)),
        ],
        out_specs=[
            pl.BlockSpec((1, 2 * _A), lambda i: (0, 0)),
            pl.BlockSpec((1, 2 * _A), lambda i: (0, 0)),
        ],
        out_shape=[
            jax.ShapeDtypeStruct((1, 2 * _A), jnp.float32),
            jax.ShapeDtypeStruct((1, 2 * _A), jnp.float32),
        ],
        scratch_shapes=[
            pltpu.VMEM((_A, _A), jnp.float32),
            pltpu.VMEM((_A, _A), jnp.float32),
            pltpu.VMEM((_A, _A), jnp.float32),
            pltpu.VMEM((_A, _NBR), jnp.float32),
            pltpu.VMEM((_A, _NBR), jnp.float32),
            pltpu.VMEM((_NBR, _NBR), jnp.float32),
            pltpu.VMEM((1, _A), jnp.float32),
            pltpu.VMEM((1, _A), jnp.float32),
            pltpu.VMEM((1, _NBR), jnp.float32),
        ],
    )(x_bf, gath3, nbr_bf, w[:_A], w[_A:2 * _A], w[2 * _A:],
      b.reshape(1, 2 * _A), g1v.reshape(1, 2 * _A), b1v.reshape(1, 2 * _A))


def _conv_pass2(x, gath3, nbr_fea, wf, bf):
    def body(x_ref, g_ref, e_ref, w1_ref, w2_ref, w3_ref, b_ref, o_ref, st_ref):
        gated = _gated_tile(x_ref, g_ref, e_ref, w1_ref, w2_ref, w3_ref, b_ref)
        filt = _sigmoid(gated[:, :_A])
        core = _softplus(gated[:, _A:])
        prod = (filt * core).reshape(_T, _M, _A)
        ns = jnp.sum(prod, axis=1)
        o_ref[...] = ns
        s = jnp.sum(ns, axis=0).reshape(1, _A)
        ss = jnp.sum(ns * ns, axis=0).reshape(1, _A)
        part = jnp.concatenate([s, ss, jnp.zeros((6, _A), jnp.float32)], axis=0)

        @pl.when(pl.program_id(0) == 0)
        def _():
            st_ref[...] = jnp.zeros_like(st_ref)

        st_ref[...] += part

    return pl.pallas_call(
        body,
        grid=(_GRID,),
        in_specs=[
            pl.BlockSpec((_T, _A), lambda i: (i, 0)),
            pl.BlockSpec((_T, _M, _A), lambda i: (i, 0, 0)),
            pl.BlockSpec((_T, _M, _NBR), lambda i: (i, 0, 0)),
            pl.BlockSpec((_A, 2 * _A), lambda i: (0, 0)),
            pl.BlockSpec((_A, 2 * _A), lambda i: (0, 0)),
            pl.BlockSpec((_NBR, 2 * _A), lambda i: (0, 0)),
            pl.BlockSpec((1, 2 * _A), lambda i: (0, 0)),
        ],
        out_specs=[
            pl.BlockSpec((_T, _A), lambda i: (i, 0)),
            pl.BlockSpec((8, _A), lambda i: (0, 0)),
        ],
        out_shape=[
            jax.ShapeDtypeStruct((_N, _A), jnp.float32),
            jax.ShapeDtypeStruct((8, _A), jnp.float32),
        ],
    )(x, gath3, nbr_fea, wf[:_A], wf[_A:2 * _A], wf[2 * _A:], bf)


def _bn_resid(x, ns, scale, shift):
    tm = 2000

    def body(x_ref, n_ref, sc_ref, sh_ref, o_ref):
        o_ref[...] = _softplus(x_ref[...] + n_ref[...] * sc_ref[...] + sh_ref[...])

    return pl.pallas_call(
        body,
        grid=(_N // tm,),
        in_specs=[
            pl.BlockSpec((tm, _A), lambda i: (i, 0)),
            pl.BlockSpec((tm, _A), lambda i: (i, 0)),
            pl.BlockSpec((1, _A), lambda i: (0, 0)),
            pl.BlockSpec((1, _A), lambda i: (0, 0)),
        ],
        out_specs=pl.BlockSpec((tm, _A), lambda i: (i, 0)),
        out_shape=jax.ShapeDtypeStruct((_N, _A), jnp.float32),
    )(x, ns, scale, shift)


def _pool_head(x3, t2, cw, cb, f0w, f0b, f1w, f1b, ow, ob):
    apc = _N // _B

    def body(x_ref, t_ref, cw_ref, cb_ref, f0w_ref, f0b_ref, f1w_ref, f1b_ref,
             ow_ref, ob_ref, o_ref):
        xv = x_ref[...]
        tv = t_ref[...]
        pools = []
        for eid in (_NI, _CU):
            mask = (tv == eid).astype(jnp.float32)
            cnt = jnp.sum(mask, axis=1)
            ssum = jnp.sum(xv * mask[:, :, None], axis=1)
            pooled = jnp.where(
                cnt[:, None] > 0, ssum / jnp.maximum(cnt, 1.0)[:, None], 0.0
            )
            pools.append(pooled)
        crys = _softplus(jnp.concatenate(pools, axis=1))
        crys = _softplus(
            jnp.dot(crys, cw_ref[...], preferred_element_type=jnp.float32)
            + cb_ref[...]
        )
        crys = _softplus(
            jnp.dot(crys, f0w_ref[...], preferred_element_type=jnp.float32)
            + f0b_ref[...]
        )
        crys = _softplus(
            jnp.dot(crys, f1w_ref[...], preferred_element_type=jnp.float32)
            + f1b_ref[...]
        )
        o_ref[...] = (
            jnp.dot(crys, ow_ref[...], preferred_element_type=jnp.float32)
            + ob_ref[...]
        )

    return pl.pallas_call(
        body,
        grid=(1,),
        in_specs=[
            pl.BlockSpec((_B, apc, _A), lambda i: (0, 0, 0)),
            pl.BlockSpec((_B, apc), lambda i: (0, 0)),
            pl.BlockSpec((2 * _A, _H), lambda i: (0, 0)),
            pl.BlockSpec((1, _H), lambda i: (0, 0)),
            pl.BlockSpec((_H, _H), lambda i: (0, 0)),
            pl.BlockSpec((1, _H), lambda i: (0, 0)),
            pl.BlockSpec((_H, _H), lambda i: (0, 0)),
            pl.BlockSpec((1, _H), lambda i: (0, 0)),
            pl.BlockSpec((_H, 1), lambda i: (0, 0)),
            pl.BlockSpec((1, 1), lambda i: (0, 0)),
        ],
        out_specs=pl.BlockSpec((_B, 1), lambda i: (0, 0)),
        out_shape=jax.ShapeDtypeStruct((_B, 1), jnp.float32),
    )(x3, t2, cw, cb, f0w, f0b, f1w, f1b, ow, ob)


def kernel(atom_fea, nbr_fea, nbr_fea_idx, crystal_atom_idx, atom_types,
           emb_W, emb_b, fc_W, fc_b, bn1_g, bn1_b, bn2_g, bn2_b,
           ctf_W, ctf_b, fcs_W, fcs_b, out_W, out_b):
    idx_flat = nbr_fea_idx.reshape(_E).astype(jnp.int32)
    nbr_bf = nbr_fea.astype(jnp.bfloat16)
    x = _embed(atom_fea, emb_W, emb_b)
    n1 = jnp.float32(_E)
    n2 = jnp.float32(_N)
    for i in range(_NCONV):
        x_bf = x.astype(jnp.bfloat16)
        gath3 = _sc_gather(idx_flat, x).reshape(_N, _M, _A)
        w, b = fc_W[i], fc_b[i]
        sc1, bfold = _gram_stats(x_bf, gath3, nbr_bf, w, b, bn1_g[i], bn1_b[i])
        wf = (w * sc1).astype(jnp.bfloat16)
        ns, st2 = _conv_pass2(x_bf, gath3, nbr_bf, wf, bfold)
        mu2 = st2[0] / n2
        var2 = st2[1] / n2 - mu2 * mu2
        sc2 = bn2_g[i] / jnp.sqrt(var2 + 1e-5)
        sh2 = bn2_b[i] - mu2 * sc2
        x = _bn_resid(x, ns, sc2.reshape(1, _A), sh2.reshape(1, _A))
    x3 = x.reshape(_B, _N // _B, _A)
    t2 = atom_types.reshape(_B, _N // _B).astype(jnp.int32)
    return _pool_head(
        x3, t2, ctf_W, ctf_b.reshape(1, _H),
        fcs_W[0], fcs_b[0].reshape(1, _H), fcs_W[1], fcs_b[1].reshape(1, _H),
        out_W, out_b.reshape(1, 1),
    )


# speedup vs baseline: 1.1063x; 1.1063x over previous
import functools

import jax
import jax.numpy as jnp
import numpy as np
from jax import lax
from jax.experimental import pallas as pl
from jax.experimental.pallas import tpu as pltpu
from jax.experimental.pallas import tpu_sc as plsc

_N = 10000
_M = 32
_A = 128
_NBR = 16
_NCONV = 3
_H = 192
_B = 100
_NI = 28
_CU = 29
_E = _N * _M
_NW = 32
_PW = _E // _NW
_CH = 80
_NCH = _PW // _CH

_T = 80
_GRID = _N // _T


def _softplus(x):
    return jnp.maximum(x, 0.0) + jnp.log(1.0 + jnp.exp(-jnp.abs(x)))


def _sigmoid(x):
    return 1.0 / (1.0 + jnp.exp(-x))


def _sc_gather(idx_flat, table):
    mesh = plsc.VectorSubcoreMesh(core_axis_name="c", subcore_axis_name="s")

    @functools.partial(
        pl.kernel,
        out_type=jax.ShapeDtypeStruct((_E, _A), jnp.float32),
        mesh=mesh,
        scratch_types=[
            pltpu.VMEM((_PW,), jnp.int32),
            pltpu.VMEM((_CH, _A), jnp.float32),
            pltpu.VMEM((_CH, _A), jnp.float32),
            pltpu.VMEM_SHARED((_N, _A), jnp.float32),
            pltpu.SemaphoreType.DMA,
            pltpu.SemaphoreType.DMA,
        ],
    )
    def gk(idx_hbm, tab_hbm, out_hbm, idx_v, rows_a, rows_b, tab_s, g0, g1):
        wid = lax.axis_index("s") * 2 + lax.axis_index("c")
        sid = lax.axis_index("s")
        base = wid * _PW

        @pl.when(sid < 10)
        def _():
            pltpu.sync_copy(
                tab_hbm.at[pl.ds(sid * 1000, 1000)],
                tab_s.at[pl.ds(sid * 1000, 1000)],
            )
        pltpu.sync_copy(idx_hbm.at[pl.ds(base, _PW)], idx_v)
        plsc.subcore_barrier()
        gsem = (g0, g1)
        bufs = (rows_a, rows_b)

        def gather(c, b):
            pltpu.async_copy(
                tab_s.at[idx_v.at[pl.ds(c * _CH, _CH)]], bufs[b], gsem[b]
            )

        def gather_wait(c, b):
            pltpu.make_async_copy(
                tab_s.at[idx_v.at[pl.ds(c * _CH, _CH)]], bufs[b], gsem[b]
            ).wait()

        gather(0, 0)

        def outer(o, carry):
            c0 = o * 2
            for b in (0, 1):
                c = c0 + b

                @pl.when(c + 1 < _NCH)
                def _():
                    gather(c + 1, 1 - b)

                @pl.when(c < _NCH)
                def _():
                    gather_wait(c, b)
                    pltpu.sync_copy(
                        bufs[b], out_hbm.at[pl.ds(base + c * _CH, _CH)]
                    )

            return carry

        lax.fori_loop(0, (_NCH + 1) // 2, outer, 0)

    return gk(idx_flat, table)


def _embed(atom_fea, emb_W, emb_b):
    tm = 2000

    def body(x_ref, w_ref, b_ref, o_ref):
        o_ref[...] = (
            jnp.dot(x_ref[...], w_ref[...], preferred_element_type=jnp.float32)
            + b_ref[...]
        )

    return pl.pallas_call(
        body,
        grid=(_N // tm,),
        in_specs=[
            pl.BlockSpec((tm, _A), lambda i: (i, 0)),
            pl.BlockSpec((_A, _A), lambda i: (0, 0)),
            pl.BlockSpec((1, _A), lambda i: (0, 0)),
        ],
        out_specs=pl.BlockSpec((tm, _A), lambda i: (i, 0)),
        out_shape=jax.ShapeDtypeStruct((_N, _A), jnp.float32),
    )(atom_fea, emb_W, emb_b.reshape(1, _A))


def _gated_tile(x_ref, g_ref, e_ref, w1_ref, w2_ref, w3_ref, b_ref):
    xw = jnp.dot(x_ref[...], w1_ref[...], preferred_element_type=jnp.float32)
    g2 = g_ref[...].reshape(_T * _M, _A).astype(jnp.bfloat16)
    gw = jnp.dot(g2, w2_ref[...], preferred_element_type=jnp.float32)
    e2 = e_ref[...].reshape(_T * _M, _NBR)
    ew = jnp.dot(e2, w3_ref[...], preferred_element_type=jnp.float32)
    xrep = jnp.broadcast_to(xw[:, None, :], (_T, _M, 2 * _A)).reshape(_T * _M, 2 * _A)
    return gw + ew + b_ref[...] + xrep


def _conv_stats(x, gath3, nbr_fea, w, b):
    def body(x_ref, g_ref, e_ref, w1_ref, w2_ref, w3_ref, b_ref, o_ref):
        gated = _gated_tile(x_ref, g_ref, e_ref, w1_ref, w2_ref, w3_ref, b_ref)
        s = jnp.sum(gated, axis=0).reshape(1, 2 * _A)
        ss = jnp.sum(gated * gated, axis=0).reshape(1, 2 * _A)
        part = jnp.concatenate([s, ss, jnp.zeros((6, 2 * _A), jnp.float32)], axis=0)

        @pl.when(pl.program_id(0) == 0)
        def _():
            o_ref[...] = jnp.zeros_like(o_ref)

        o_ref[...] += part

    return pl.pallas_call(
        body,
        grid=(_GRID,),
        in_specs=[
            pl.BlockSpec((_T, _A), lambda i: (i, 0)),
            pl.BlockSpec((_T, _M, _A), lambda i: (i, 0, 0)),
            pl.BlockSpec((_T, _M, _NBR), lambda i: (i, 0, 0)),
            pl.BlockSpec((_A, 2 * _A), lambda i: (0, 0)),
            pl.BlockSpec((_A, 2 * _A), lambda i: (0, 0)),
            pl.BlockSpec((_NBR, 2 * _A), lambda i: (0, 0)),
            pl.BlockSpec((1, 2 * _A), lambda i: (0, 0)),
        ],
        out_specs=pl.BlockSpec((8, 2 * _A), lambda i: (0, 0)),
        out_shape=jax.ShapeDtypeStruct((8, 2 * _A), jnp.float32),
    )(x, gath3, nbr_fea, w[:_A], w[_A:2 * _A], w[2 * _A:], b.reshape(1, 2 * _A))


def _gram_stats(x_bf, gath3, nbr_bf, w, b, g1v, b1v):
    f32 = jnp.float32
    dn = (((0,), (0,)), ((), ()))

    def body(x_ref, g_ref, e_ref, w1_ref, w2_ref, w3_ref, bb_ref, g1_ref, b1_ref,
             sc_ref, bf_ref, p1_s, p3_s, c22_s, c23_s, p4_s, q_s, vx_s, vg_s, ve_s):
        i = pl.program_id(0)
        xt = x_ref[...]
        g32 = g_ref[...]
        gt = g32.astype(jnp.bfloat16).reshape(_T * _M, _A)
        et = e_ref[...].reshape(_T * _M, _NBR)
        st32 = jnp.sum(g32, axis=1)
        est32 = jnp.sum(e_ref[...].astype(f32), axis=1)
        stb = st32.astype(jnp.bfloat16)
        estb = est32.astype(jnp.bfloat16)

        @pl.when(i == 0)
        def _():
            p1_s[...] = jnp.zeros_like(p1_s)
            p3_s[...] = jnp.zeros_like(p3_s)
            c22_s[...] = jnp.zeros_like(c22_s)
            c23_s[...] = jnp.zeros_like(c23_s)
            p4_s[...] = jnp.zeros_like(p4_s)
            q_s[...] = jnp.zeros_like(q_s)
            vx_s[...] = jnp.zeros_like(vx_s)
            vg_s[...] = jnp.zeros_like(vg_s)
            ve_s[...] = jnp.zeros_like(ve_s)

        p1_s[...] += lax.dot_general(xt, xt, dn, preferred_element_type=f32)
        p3_s[...] += lax.dot_general(xt, stb, dn, preferred_element_type=f32)
        c22_s[...] += lax.dot_general(gt, gt, dn, preferred_element_type=f32)
        c23_s[...] += lax.dot_general(gt, et, dn, preferred_element_type=f32)
        p4_s[...] += lax.dot_general(xt, estb, dn, preferred_element_type=f32)
        q_s[...] += lax.dot_general(et, et, dn, preferred_element_type=f32)
        vx_s[...] += jnp.sum(xt.astype(f32), axis=0).reshape(1, _A)
        vg_s[...] += jnp.sum(st32, axis=0).reshape(1, _A)
        ve_s[...] += jnp.sum(est32, axis=0).reshape(1, _NBR)

        @pl.when(i == _GRID - 1)
        def _():
            w1 = w1_ref[...]
            w2 = w2_ref[...]
            w3 = w3_ref[...]
            bb = bb_ref[...]
            n = f32(_E)
            c11 = p1_s[...] * f32(_M)
            t1 = jnp.sum(w1 * jnp.dot(c11, w1, preferred_element_type=f32), axis=0)
            t2 = jnp.sum(w2 * jnp.dot(c22_s[...], w2, preferred_element_type=f32), axis=0)
            t3 = jnp.sum(w3 * jnp.dot(q_s[...], w3, preferred_element_type=f32), axis=0)
            c1 = jnp.sum(w1 * jnp.dot(p3_s[...], w2, preferred_element_type=f32), axis=0)
            c2 = jnp.sum(w1 * jnp.dot(p4_s[...], w3, preferred_element_type=f32), axis=0)
            c3 = jnp.sum(w2 * jnp.dot(c23_s[...], w3, preferred_element_type=f32), axis=0)
            d = (t1 + t2 + t3 + 2.0 * (c1 + c2 + c3)).reshape(1, 2 * _A)
            sdot = (
                f32(_M) * jnp.dot(vx_s[...], w1, preferred_element_type=f32)
                + jnp.dot(vg_s[...], w2, preferred_element_type=f32)
                + jnp.dot(ve_s[...], w3, preferred_element_type=f32)
                + n * bb
            )
            mu = sdot / n
            sumsq = d + 2.0 * bb * sdot - n * bb * bb
            var = sumsq / n - mu * mu
            sc = g1_ref[...] / jnp.sqrt(var + 1e-5)
            sc_ref[...] = sc
            bf_ref[...] = (bb - mu) * sc + b1_ref[...]

    return pl.pallas_call(
        body,
        grid=(_GRID,),
        in_specs=[
            pl.BlockSpec((_T, _A), lambda i: (i, 0)),
            pl.BlockSpec((_T, _M, _A), lambda i: (i, 0, 0)),
            pl.BlockSpec((_T, _M, _NBR), lambda i: (i, 0, 0)),
            pl.BlockSpec((_A, 2 * _A), lambda i: (0, 0)),
            pl.BlockSpec((_A, 2 * _A), lambda i: (0, 0)),
            pl.BlockSpec((_NBR, 2 * _A), lambda i: (0, 0)),
            pl.BlockSpec((1, 2 * _A), lambda i: (0, 0)),
            pl.BlockSpec((1, 2 * _A), lambda i: (0, 0)),
            pl.BlockSpec((1, 2 * _A), lambda i: (0, 0)),
        ],
        out_specs=[
            pl.BlockSpec((1, 2 * _A), lambda i: (0, 0)),
            pl.BlockSpec((1, 2 * _A), lambda i: (0, 0)),
        ],
        out_shape=[
            jax.ShapeDtypeStruct((1, 2 * _A), jnp.float32),
            jax.ShapeDtypeStruct((1, 2 * _A), jnp.float32),
        ],
        scratch_shapes=[
            pltpu.VMEM((_A, _A), jnp.float32),
            pltpu.VMEM((_A, _A), jnp.float32),
            pltpu.VMEM((_A, _A), jnp.float32),
            pltpu.VMEM((_A, _NBR), jnp.float32),
            pltpu.VMEM((_A, _NBR), jnp.float32),
            pltpu.VMEM((_NBR, _NBR), jnp.float32),
            pltpu.VMEM((1, _A), jnp.float32),
            pltpu.VMEM((1, _A), jnp.float32),
            pltpu.VMEM((1, _NBR), jnp.float32),
        ],
    )(x_bf, gath3, nbr_bf, w[:_A], w[_A:2 * _A], w[2 * _A:],
      b.reshape(1, 2 * _A), g1v.reshape(1, 2 * _A), b1v.reshape(1, 2 * _A))


def _conv_pass2(x, gath3, nbr_fea, wf, bf):
    def body(x_ref, g_ref, e_ref, w1_ref, w2_ref, w3_ref, b_ref, o_ref, st_ref):
        gated = _gated_tile(x_ref, g_ref, e_ref, w1_ref, w2_ref, w3_ref, b_ref)
        filt = _sigmoid(gated[:, :_A])
        core = _softplus(gated[:, _A:])
        prod = (filt * core).reshape(_T, _M, _A)
        ns = jnp.sum(prod, axis=1)
        o_ref[...] = ns
        s = jnp.sum(ns, axis=0).reshape(1, _A)
        ss = jnp.sum(ns * ns, axis=0).reshape(1, _A)
        part = jnp.concatenate([s, ss, jnp.zeros((6, _A), jnp.float32)], axis=0)

        @pl.when(pl.program_id(0) == 0)
        def _():
            st_ref[...] = jnp.zeros_like(st_ref)

        st_ref[...] += part

    return pl.pallas_call(
        body,
        grid=(_GRID,),
        in_specs=[
            pl.BlockSpec((_T, _A), lambda i: (i, 0)),
            pl.BlockSpec((_T, _M, _A), lambda i: (i, 0, 0)),
            pl.BlockSpec((_T, _M, _NBR), lambda i: (i, 0, 0)),
            pl.BlockSpec((_A, 2 * _A), lambda i: (0, 0)),
            pl.BlockSpec((_A, 2 * _A), lambda i: (0, 0)),
            pl.BlockSpec((_NBR, 2 * _A), lambda i: (0, 0)),
            pl.BlockSpec((1, 2 * _A), lambda i: (0, 0)),
        ],
        out_specs=[
            pl.BlockSpec((_T, _A), lambda i: (i, 0)),
            pl.BlockSpec((8, _A), lambda i: (0, 0)),
        ],
        out_shape=[
            jax.ShapeDtypeStruct((_N, _A), jnp.float32),
            jax.ShapeDtypeStruct((8, _A), jnp.float32),
        ],
    )(x, gath3, nbr_fea, wf[:_A], wf[_A:2 * _A], wf[2 * _A:], bf)


def _bn_resid(x, ns, scale, shift):
    tm = 2000

    def body(x_ref, n_ref, sc_ref, sh_ref, o_ref):
        o_ref[...] = _softplus(x_ref[...] + n_ref[...] * sc_ref[...] + sh_ref[...])

    return pl.pallas_call(
        body,
        grid=(_N // tm,),
        in_specs=[
            pl.BlockSpec((tm, _A), lambda i: (i, 0)),
            pl.BlockSpec((tm, _A), lambda i: (i, 0)),
            pl.BlockSpec((1, _A), lambda i: (0, 0)),
            pl.BlockSpec((1, _A), lambda i: (0, 0)),
        ],
        out_specs=pl.BlockSpec((tm, _A), lambda i: (i, 0)),
        out_shape=jax.ShapeDtypeStruct((_N, _A), jnp.float32),
    )(x, ns, scale, shift)


def _pool_head(x3, t2, cw, cb, f0w, f0b, f1w, f1b, ow, ob):
    apc = _N // _B

    def body(x_ref, t_ref, cw_ref, cb_ref, f0w_ref, f0b_ref, f1w_ref, f1b_ref,
             ow_ref, ob_ref, o_ref):
        xv = x_ref[...]
        tv = t_ref[...]
        pools = []
        for eid in (_NI, _CU):
            mask = (tv == eid).astype(jnp.float32)
            cnt = jnp.sum(mask, axis=1)
            ssum = jnp.sum(xv * mask[:, :, None], axis=1)
            pooled = jnp.where(
                cnt[:, None] > 0, ssum / jnp.maximum(cnt, 1.0)[:, None], 0.0
            )
            pools.append(pooled)
        crys = _softplus(jnp.concatenate(pools, axis=1))
        crys = _softplus(
            jnp.dot(crys, cw_ref[...], preferred_element_type=jnp.float32)
            + cb_ref[...]
        )
        crys = _softplus(
            jnp.dot(crys, f0w_ref[...], preferred_element_type=jnp.float32)
            + f0b_ref[...]
        )
        crys = _softplus(
            jnp.dot(crys, f1w_ref[...], preferred_element_type=jnp.float32)
            + f1b_ref[...]
        )
        o_ref[...] = (
            jnp.dot(crys, ow_ref[...], preferred_element_type=jnp.float32)
            + ob_ref[...]
        )

    return pl.pallas_call(
        body,
        grid=(1,),
        in_specs=[
            pl.BlockSpec((_B, apc, _A), lambda i: (0, 0, 0)),
            pl.BlockSpec((_B, apc), lambda i: (0, 0)),
            pl.BlockSpec((2 * _A, _H), lambda i: (0, 0)),
            pl.BlockSpec((1, _H), lambda i: (0, 0)),
            pl.BlockSpec((_H, _H), lambda i: (0, 0)),
            pl.BlockSpec((1, _H), lambda i: (0, 0)),
            pl.BlockSpec((_H, _H), lambda i: (0, 0)),
            pl.BlockSpec((1, _H), lambda i: (0, 0)),
            pl.BlockSpec((_H, 1), lambda i: (0, 0)),
            pl.BlockSpec((1, 1), lambda i: (0, 0)),
        ],
        out_specs=pl.BlockSpec((_B, 1), lambda i: (0, 0)),
        out_shape=jax.ShapeDtypeStruct((_B, 1), jnp.float32),
    )(x3, t2, cw, cb, f0w, f0b, f1w, f1b, ow, ob)


def kernel(atom_fea, nbr_fea, nbr_fea_idx, crystal_atom_idx, atom_types,
           emb_W, emb_b, fc_W, fc_b, bn1_g, bn1_b, bn2_g, bn2_b,
           ctf_W, ctf_b, fcs_W, fcs_b, out_W, out_b):
    idx_flat = nbr_fea_idx.reshape(_E).astype(jnp.int32)
    nbr_bf = nbr_fea.astype(jnp.bfloat16)
    x = _embed(atom_fea, emb_W, emb_b)
    n1 = jnp.float32(_E)
    n2 = jnp.float32(_N)
    for i in range(_NCONV):
        x_bf = x.astype(jnp.bfloat16)
        gath3 = _sc_gather(idx_flat, x).reshape(_N, _M, _A)
        w, b = fc_W[i], fc_b[i]
        sc1, bfold = _gram_stats(x_bf, gath3, nbr_bf, w, b, bn1_g[i], bn1_b[i])
        wf = (w * sc1).astype(jnp.bfloat16)
        ns, st2 = _conv_pass2(x_bf, gath3, nbr_bf, wf, bfold)
        mu2 = st2[0] / n2
        var2 = st2[1] / n2 - mu2 * mu2
        sc2 = bn2_g[i] / jnp.sqrt(var2 + 1e-5)
        sh2 = bn2_b[i] - mu2 * sc2
        x = _bn_resid(x, ns, sc2.reshape(1, _A), sh2.reshape(1, _A))
    x3 = x.reshape(_B, _N // _B, _A)
    t2 = atom_types.reshape(_B, _N // _B).astype(jnp.int32)
    return _pool_head(
        x3, t2, ctf_W, ctf_b.reshape(1, _H),
        fcs_W[0], fcs_b[0].reshape(1, _H), fcs_W[1], fcs_b[1].reshape(1, _H),
        out_W, out_b.reshape(1, 1),
    )


# speedup vs baseline: 1.5054x; 1.3607x over previous
import functools

import jax
import jax.numpy as jnp
import numpy as np
from jax import lax
from jax.experimental import pallas as pl
from jax.experimental.pallas import tpu as pltpu
from jax.experimental.pallas import tpu_sc as plsc

_N = 10000
_M = 32
_A = 128
_NBR = 16
_NCONV = 3
_H = 192
_B = 100
_NI = 28
_CU = 29
_E = _N * _M
_NW = 32
_PW = _E // _NW
_CH = 80
_NCH = _PW // _CH

_T = 400
_GRID = _N // _T


def _softplus(x):
    return jnp.maximum(x, 0.0) + jnp.log(1.0 + jnp.exp(-jnp.abs(x)))


def _sigmoid(x):
    return 1.0 / (1.0 + jnp.exp(-x))


def _sc_gather(idx_flat, table):
    mesh = plsc.VectorSubcoreMesh(core_axis_name="c", subcore_axis_name="s")

    @functools.partial(
        pl.kernel,
        out_type=jax.ShapeDtypeStruct((_E, _A), jnp.float32),
        mesh=mesh,
        scratch_types=[
            pltpu.VMEM((_PW,), jnp.int32),
            pltpu.VMEM((_CH, _A), jnp.float32),
            pltpu.VMEM((_CH, _A), jnp.float32),
            pltpu.VMEM_SHARED((_N, _A), jnp.float32),
            pltpu.SemaphoreType.DMA,
            pltpu.SemaphoreType.DMA,
        ],
    )
    def gk(idx_hbm, tab_hbm, out_hbm, idx_v, rows_a, rows_b, tab_s, g0, g1):
        wid = lax.axis_index("s") * 2 + lax.axis_index("c")
        sid = lax.axis_index("s")
        base = wid * _PW

        @pl.when(sid < 10)
        def _():
            pltpu.sync_copy(
                tab_hbm.at[pl.ds(sid * 1000, 1000)],
                tab_s.at[pl.ds(sid * 1000, 1000)],
            )
        pltpu.sync_copy(idx_hbm.at[pl.ds(base, _PW)], idx_v)
        plsc.subcore_barrier()
        gsem = (g0, g1)
        bufs = (rows_a, rows_b)

        def gather(c, b):
            pltpu.async_copy(
                tab_s.at[idx_v.at[pl.ds(c * _CH, _CH)]], bufs[b], gsem[b]
            )

        def gather_wait(c, b):
            pltpu.make_async_copy(
                tab_s.at[idx_v.at[pl.ds(c * _CH, _CH)]], bufs[b], gsem[b]
            ).wait()

        gather(0, 0)

        def outer(o, carry):
            c0 = o * 2
            for b in (0, 1):
                c = c0 + b

                @pl.when(c + 1 < _NCH)
                def _():
                    gather(c + 1, 1 - b)

                @pl.when(c < _NCH)
                def _():
                    gather_wait(c, b)
                    pltpu.sync_copy(
                        bufs[b], out_hbm.at[pl.ds(base + c * _CH, _CH)]
                    )

            return carry

        lax.fori_loop(0, (_NCH + 1) // 2, outer, 0)

    return gk(idx_flat, table)


def _embed(atom_fea, emb_W, emb_b):
    tm = 2000

    def body(x_ref, w_ref, b_ref, o_ref):
        o_ref[...] = (
            jnp.dot(x_ref[...], w_ref[...], preferred_element_type=jnp.float32)
            + b_ref[...]
        )

    return pl.pallas_call(
        body,
        grid=(_N // tm,),
        in_specs=[
            pl.BlockSpec((tm, _A), lambda i: (i, 0)),
            pl.BlockSpec((_A, _A), lambda i: (0, 0)),
            pl.BlockSpec((1, _A), lambda i: (0, 0)),
        ],
        out_specs=pl.BlockSpec((tm, _A), lambda i: (i, 0)),
        out_shape=jax.ShapeDtypeStruct((_N, _A), jnp.float32),
    )(atom_fea, emb_W, emb_b.reshape(1, _A))


def _gated_tile(x_ref, g_ref, e_ref, w1_ref, w23_ref, b_ref):
    xw = jnp.dot(x_ref[...], w1_ref[...], preferred_element_type=jnp.float32)
    g2 = g_ref[...].reshape(_T * _M, _A).astype(jnp.bfloat16)
    e2 = e_ref[...].reshape(_T * _M, _NBR)
    ge = jnp.concatenate([g2, e2], axis=1)
    gew = jnp.dot(ge, w23_ref[...], preferred_element_type=jnp.float32)
    xrep = jnp.broadcast_to(xw[:, None, :], (_T, _M, 2 * _A)).reshape(_T * _M, 2 * _A)
    return gew + b_ref[...] + xrep


def _gram_stats(x_bf, gath3, nbr_bf, w, b, g1v, b1v):
    f32 = jnp.float32
    dn = (((0,), (0,)), ((), ()))

    def body(x_ref, g_ref, e_ref, w1_ref, w2_ref, w3_ref, bb_ref, g1_ref, b1_ref,
             sc_ref, bf_ref, xz_s, zz_s, vx_s, vg_s, ve_s):
        i = pl.program_id(0)
        xt = x_ref[...]
        g32 = g_ref[...]
        gt = g32.astype(jnp.bfloat16).reshape(_T * _M, _A)
        et = e_ref[...].reshape(_T * _M, _NBR)
        st32 = jnp.sum(g32, axis=1)
        est32 = jnp.sum(e_ref[...].astype(f32), axis=1)
        stb = st32.astype(jnp.bfloat16)
        estb = est32.astype(jnp.bfloat16)

        @pl.when(i == 0)
        def _():
            xz_s[...] = jnp.zeros_like(xz_s)
            zz_s[...] = jnp.zeros_like(zz_s)
            vx_s[...] = jnp.zeros_like(vx_s)
            vg_s[...] = jnp.zeros_like(vg_s)
            ve_s[...] = jnp.zeros_like(ve_s)

        ze = jnp.concatenate([gt, et], axis=1)
        xs = jnp.concatenate([xt, stb, estb], axis=1)
        xz_s[...] += lax.dot_general(xt, xs, dn, preferred_element_type=f32)
        zz_s[...] += lax.dot_general(ze, ze, dn, preferred_element_type=f32)
        vx_s[...] += jnp.sum(xt.astype(f32), axis=0).reshape(1, _A)
        vg_s[...] += jnp.sum(st32, axis=0).reshape(1, _A)
        ve_s[...] += jnp.sum(est32, axis=0).reshape(1, _NBR)

        @pl.when(i == _GRID - 1)
        def _():
            w1 = w1_ref[...]
            w2 = w2_ref[...]
            w3 = w3_ref[...]
            bb = bb_ref[...]
            n = f32(_E)
            xz = xz_s[...]
            zz = zz_s[...]
            c11 = xz[:, :_A] * f32(_M)
            c12 = xz[:, _A:2 * _A]
            c13 = xz[:, 2 * _A:]
            c22 = zz[:_A, :_A]
            c23 = zz[:_A, _A:]
            q = zz[_A:, _A:]
            t1 = jnp.sum(w1 * jnp.dot(c11, w1, preferred_element_type=f32), axis=0)
            t2 = jnp.sum(w2 * jnp.dot(c22, w2, preferred_element_type=f32), axis=0)
            t3 = jnp.sum(w3 * jnp.dot(q, w3, preferred_element_type=f32), axis=0)
            c1 = jnp.sum(w1 * jnp.dot(c12, w2, preferred_element_type=f32), axis=0)
            c2 = jnp.sum(w1 * jnp.dot(c13, w3, preferred_element_type=f32), axis=0)
            c3 = jnp.sum(w2 * jnp.dot(c23, w3, preferred_element_type=f32), axis=0)
            d = (t1 + t2 + t3 + 2.0 * (c1 + c2 + c3)).reshape(1, 2 * _A)
            sdot = (
                f32(_M) * jnp.dot(vx_s[...], w1, preferred_element_type=f32)
                + jnp.dot(vg_s[...], w2, preferred_element_type=f32)
                + jnp.dot(ve_s[...], w3, preferred_element_type=f32)
                + n * bb
            )
            mu = sdot / n
            sumsq = d + 2.0 * bb * sdot - n * bb * bb
            var = sumsq / n - mu * mu
            sc = g1_ref[...] / jnp.sqrt(var + 1e-5)
            sc_ref[...] = sc
            bf_ref[...] = (bb - mu) * sc + b1_ref[...]

    return pl.pallas_call(
        body,
        grid=(_GRID,),
        in_specs=[
            pl.BlockSpec((_T, _A), lambda i: (i, 0)),
            pl.BlockSpec((_T, _M, _A), lambda i: (i, 0, 0)),
            pl.BlockSpec((_T, _M, _NBR), lambda i: (i, 0, 0)),
            pl.BlockSpec((_A, 2 * _A), lambda i: (0, 0)),
            pl.BlockSpec((_A, 2 * _A), lambda i: (0, 0)),
            pl.BlockSpec((_NBR, 2 * _A), lambda i: (0, 0)),
            pl.BlockSpec((1, 2 * _A), lambda i: (0, 0)),
            pl.BlockSpec((1, 2 * _A), lambda i: (0, 0)),
            pl.BlockSpec((1, 2 * _A), lambda i: (0, 0)),
        ],
        out_specs=[
            pl.BlockSpec((1, 2 * _A), lambda i: (0, 0)),
            pl.BlockSpec((1, 2 * _A), lambda i: (0, 0)),
        ],
        out_shape=[
            jax.ShapeDtypeStruct((1, 2 * _A), jnp.float32),
            jax.ShapeDtypeStruct((1, 2 * _A), jnp.float32),
        ],
        scratch_shapes=[
            pltpu.VMEM((_A, 2 * _A + _NBR), jnp.float32),
            pltpu.VMEM((_A + _NBR, _A + _NBR), jnp.float32),
            pltpu.VMEM((1, _A), jnp.float32),
            pltpu.VMEM((1, _A), jnp.float32),
            pltpu.VMEM((1, _NBR), jnp.float32),
        ],
    )(x_bf, gath3, nbr_bf, w[:_A], w[_A:2 * _A], w[2 * _A:],
      b.reshape(1, 2 * _A), g1v.reshape(1, 2 * _A), b1v.reshape(1, 2 * _A))


def _conv_pass2(x, gath3, nbr_fea, wf, bf):
    def body(x_ref, g_ref, e_ref, w1_ref, w23_ref, b_ref, o_ref, st_ref):
        gated = _gated_tile(x_ref, g_ref, e_ref, w1_ref, w23_ref, b_ref)
        filt = _sigmoid(gated[:, :_A])
        core = jnp.log(1.0 + jnp.exp(gated[:, _A:]))
        prod = (filt * core).reshape(_T, _M, _A)
        ns = jnp.sum(prod, axis=1)
        o_ref[...] = ns
        s = jnp.sum(ns, axis=0).reshape(1, _A)
        ss = jnp.sum(ns * ns, axis=0).reshape(1, _A)
        part = jnp.concatenate([s, ss, jnp.zeros((6, _A), jnp.float32)], axis=0)

        @pl.when(pl.program_id(0) == 0)
        def _():
            st_ref[...] = jnp.zeros_like(st_ref)

        st_ref[...] += part

    return pl.pallas_call(
        body,
        grid=(_GRID,),
        in_specs=[
            pl.BlockSpec((_T, _A), lambda i: (i, 0)),
            pl.BlockSpec((_T, _M, _A), lambda i: (i, 0, 0)),
            pl.BlockSpec((_T, _M, _NBR), lambda i: (i, 0, 0)),
            pl.BlockSpec((_A, 2 * _A), lambda i: (0, 0)),
            pl.BlockSpec((_A + _NBR, 2 * _A), lambda i: (0, 0)),
            pl.BlockSpec((1, 2 * _A), lambda i: (0, 0)),
        ],
        out_specs=[
            pl.BlockSpec((_T, _A), lambda i: (i, 0)),
            pl.BlockSpec((8, _A), lambda i: (0, 0)),
        ],
        out_shape=[
            jax.ShapeDtypeStruct((_N, _A), jnp.float32),
            jax.ShapeDtypeStruct((8, _A), jnp.float32),
        ],
    )(x, gath3, nbr_fea, wf[:_A], wf[_A:], bf)


def _bn_resid(x, ns, scale, shift):
    tm = 2000

    def body(x_ref, n_ref, sc_ref, sh_ref, o_ref):
        o_ref[...] = _softplus(x_ref[...] + n_ref[...] * sc_ref[...] + sh_ref[...])

    return pl.pallas_call(
        body,
        grid=(_N // tm,),
        in_specs=[
            pl.BlockSpec((tm, _A), lambda i: (i, 0)),
            pl.BlockSpec((tm, _A), lambda i: (i, 0)),
            pl.BlockSpec((1, _A), lambda i: (0, 0)),
            pl.BlockSpec((1, _A), lambda i: (0, 0)),
        ],
        out_specs=pl.BlockSpec((tm, _A), lambda i: (i, 0)),
        out_shape=jax.ShapeDtypeStruct((_N, _A), jnp.float32),
    )(x, ns, scale, shift)


def _pool_head(x3, t2, cw, cb, f0w, f0b, f1w, f1b, ow, ob):
    apc = _N // _B

    def body(x_ref, t_ref, cw_ref, cb_ref, f0w_ref, f0b_ref, f1w_ref, f1b_ref,
             ow_ref, ob_ref, o_ref):
        xv = x_ref[...]
        tv = t_ref[...]
        pools = []
        for eid in (_NI, _CU):
            mask = (tv == eid).astype(jnp.float32)
            cnt = jnp.sum(mask, axis=1)
            ssum = jnp.sum(xv * mask[:, :, None], axis=1)
            pooled = jnp.where(
                cnt[:, None] > 0, ssum / jnp.maximum(cnt, 1.0)[:, None], 0.0
            )
            pools.append(pooled)
        crys = _softplus(jnp.concatenate(pools, axis=1))
        crys = _softplus(
            jnp.dot(crys, cw_ref[...], preferred_element_type=jnp.float32)
            + cb_ref[...]
        )
        crys = _softplus(
            jnp.dot(crys, f0w_ref[...], preferred_element_type=jnp.float32)
            + f0b_ref[...]
        )
        crys = _softplus(
            jnp.dot(crys, f1w_ref[...], preferred_element_type=jnp.float32)
            + f1b_ref[...]
        )
        o_ref[...] = (
            jnp.dot(crys, ow_ref[...], preferred_element_type=jnp.float32)
            + ob_ref[...]
        )

    return pl.pallas_call(
        body,
        grid=(1,),
        in_specs=[
            pl.BlockSpec((_B, apc, _A), lambda i: (0, 0, 0)),
            pl.BlockSpec((_B, apc), lambda i: (0, 0)),
            pl.BlockSpec((2 * _A, _H), lambda i: (0, 0)),
            pl.BlockSpec((1, _H), lambda i: (0, 0)),
            pl.BlockSpec((_H, _H), lambda i: (0, 0)),
            pl.BlockSpec((1, _H), lambda i: (0, 0)),
            pl.BlockSpec((_H, _H), lambda i: (0, 0)),
            pl.BlockSpec((1, _H), lambda i: (0, 0)),
            pl.BlockSpec((_H, 1), lambda i: (0, 0)),
            pl.BlockSpec((1, 1), lambda i: (0, 0)),
        ],
        out_specs=pl.BlockSpec((_B, 1), lambda i: (0, 0)),
        out_shape=jax.ShapeDtypeStruct((_B, 1), jnp.float32),
    )(x3, t2, cw, cb, f0w, f0b, f1w, f1b, ow, ob)


def kernel(atom_fea, nbr_fea, nbr_fea_idx, crystal_atom_idx, atom_types,
           emb_W, emb_b, fc_W, fc_b, bn1_g, bn1_b, bn2_g, bn2_b,
           ctf_W, ctf_b, fcs_W, fcs_b, out_W, out_b):
    idx_flat = nbr_fea_idx.reshape(_E).astype(jnp.int32)
    nbr_bf = nbr_fea.astype(jnp.bfloat16)
    x = _embed(atom_fea, emb_W, emb_b)
    n1 = jnp.float32(_E)
    n2 = jnp.float32(_N)
    for i in range(_NCONV):
        x_bf = x.astype(jnp.bfloat16)
        gath3 = _sc_gather(idx_flat, x).reshape(_N, _M, _A)
        w, b = fc_W[i], fc_b[i]
        sc1, bfold = _gram_stats(x_bf, gath3, nbr_bf, w, b, bn1_g[i], bn1_b[i])
        wf = (w * sc1).astype(jnp.bfloat16)
        ns, st2 = _conv_pass2(x_bf, gath3, nbr_bf, wf, bfold)
        mu2 = st2[0] / n2
        var2 = st2[1] / n2 - mu2 * mu2
        sc2 = bn2_g[i] / jnp.sqrt(var2 + 1e-5)
        sh2 = bn2_b[i] - mu2 * sc2
        x = _bn_resid(x, ns, sc2.reshape(1, _A), sh2.reshape(1, _A))
    x3 = x.reshape(_B, _N // _B, _A)
    t2 = atom_types.reshape(_B, _N // _B).astype(jnp.int32)
    return _pool_head(
        x3, t2, ctf_W, ctf_b.reshape(1, _H),
        fcs_W[0], fcs_b[0].reshape(1, _H), fcs_W[1], fcs_b[1].reshape(1, _H),
        out_W, out_b.reshape(1, 1),
    )


# speedup vs baseline: 1.5264x; 1.0140x over previous
import functools

import jax
import jax.numpy as jnp
import numpy as np
from jax import lax
from jax.experimental import pallas as pl
from jax.experimental.pallas import tpu as pltpu
from jax.experimental.pallas import tpu_sc as plsc

_N = 10000
_M = 32
_A = 128
_NBR = 16
_NCONV = 3
_H = 192
_B = 100
_NI = 28
_CU = 29
_E = _N * _M
_NW = 32
_PW = _E // _NW
_CH = 80
_NCH = _PW // _CH

_T = 400
_GRID = _N // _T


def _softplus(x):
    return jnp.maximum(x, 0.0) + jnp.log(1.0 + jnp.exp(-jnp.abs(x)))


def _sigmoid(x):
    return 1.0 / (1.0 + jnp.exp(-x))


def _sc_gather(idx_flat, table):
    mesh = plsc.VectorSubcoreMesh(core_axis_name="c", subcore_axis_name="s")

    @functools.partial(
        pl.kernel,
        out_type=jax.ShapeDtypeStruct((_E, _A), jnp.float32),
        mesh=mesh,
        scratch_types=[
            pltpu.VMEM((_PW,), jnp.int32),
            pltpu.VMEM((_CH, _A), jnp.float32),
            pltpu.VMEM((_CH, _A), jnp.float32),
            pltpu.VMEM_SHARED((_N, _A), jnp.float32),
            pltpu.SemaphoreType.DMA,
            pltpu.SemaphoreType.DMA,
        ],
    )
    def gk(idx_hbm, tab_hbm, out_hbm, idx_v, rows_a, rows_b, tab_s, g0, g1):
        wid = lax.axis_index("s") * 2 + lax.axis_index("c")
        sid = lax.axis_index("s")
        base = wid * _PW

        @pl.when(sid < 10)
        def _():
            pltpu.sync_copy(
                tab_hbm.at[pl.ds(sid * 1000, 1000)],
                tab_s.at[pl.ds(sid * 1000, 1000)],
            )
        pltpu.sync_copy(idx_hbm.at[pl.ds(base, _PW)], idx_v)
        plsc.subcore_barrier()
        gsem = (g0, g1)
        bufs = (rows_a, rows_b)

        def gather(c, b):
            pltpu.async_copy(
                tab_s.at[idx_v.at[pl.ds(c * _CH, _CH)]], bufs[b], gsem[b]
            )

        def gather_wait(c, b):
            pltpu.make_async_copy(
                tab_s.at[idx_v.at[pl.ds(c * _CH, _CH)]], bufs[b], gsem[b]
            ).wait()

        gather(0, 0)

        def outer(o, carry):
            c0 = o * 2
            for b in (0, 1):
                c = c0 + b

                @pl.when(c + 1 < _NCH)
                def _():
                    gather(c + 1, 1 - b)

                @pl.when(c < _NCH)
                def _():
                    gather_wait(c, b)
                    pltpu.sync_copy(
                        bufs[b], out_hbm.at[pl.ds(base + c * _CH, _CH)]
                    )

            return carry

        lax.fori_loop(0, (_NCH + 1) // 2, outer, 0)

    return gk(idx_flat, table)


def _embed(atom_fea, emb_W, emb_b):
    tm = 2000

    def body(x_ref, w_ref, b_ref, o_ref):
        o_ref[...] = (
            jnp.dot(x_ref[...], w_ref[...], preferred_element_type=jnp.float32)
            + b_ref[...]
        )

    return pl.pallas_call(
        body,
        grid=(_N // tm,),
        in_specs=[
            pl.BlockSpec((tm, _A), lambda i: (i, 0)),
            pl.BlockSpec((_A, _A), lambda i: (0, 0)),
            pl.BlockSpec((1, _A), lambda i: (0, 0)),
        ],
        out_specs=pl.BlockSpec((tm, _A), lambda i: (i, 0)),
        out_shape=jax.ShapeDtypeStruct((_N, _A), jnp.float32),
    )(atom_fea, emb_W, emb_b.reshape(1, _A))


def _gated_tile(x_ref, g_ref, e_ref, w1_ref, w23_ref, b_ref):
    xw = jnp.dot(x_ref[...], w1_ref[...], preferred_element_type=jnp.float32)
    g2 = g_ref[...].reshape(_T * _M, _A).astype(jnp.bfloat16)
    e2 = e_ref[...].reshape(_T * _M, _NBR)
    ge = jnp.concatenate([g2, e2], axis=1)
    gew = jnp.dot(ge, w23_ref[...], preferred_element_type=jnp.float32)
    gated3 = gew.reshape(_T, _M, 2 * _A) + xw[:, None, :] + b_ref[...]
    return gated3.reshape(_T * _M, 2 * _A)


def _gram_stats(x_bf, gath3, nbr_bf, w, b, g1v, b1v):
    f32 = jnp.float32
    dn = (((0,), (0,)), ((), ()))

    def body(x_ref, g_ref, e_ref, w1_ref, w2_ref, w3_ref, bb_ref, g1_ref, b1_ref,
             sc_ref, bf_ref, xz_s, zz_s, vx_s, vg_s, ve_s):
        i = pl.program_id(0)
        xt = x_ref[...]
        g32 = g_ref[...]
        gt = g32.astype(jnp.bfloat16).reshape(_T * _M, _A)
        et = e_ref[...].reshape(_T * _M, _NBR)
        st32 = jnp.sum(g32, axis=1)
        est32 = jnp.sum(e_ref[...].astype(f32), axis=1)
        stb = st32.astype(jnp.bfloat16)
        estb = est32.astype(jnp.bfloat16)

        @pl.when(i == 0)
        def _():
            xz_s[...] = jnp.zeros_like(xz_s)
            zz_s[...] = jnp.zeros_like(zz_s)
            vx_s[...] = jnp.zeros_like(vx_s)
            vg_s[...] = jnp.zeros_like(vg_s)
            ve_s[...] = jnp.zeros_like(ve_s)

        ze = jnp.concatenate([gt, et], axis=1)
        xs = jnp.concatenate([xt, stb, estb], axis=1)
        xz_s[...] += lax.dot_general(xt, xs, dn, preferred_element_type=f32)
        zz_s[...] += lax.dot_general(ze, ze, dn, preferred_element_type=f32)
        vx_s[...] += jnp.sum(xt.astype(f32), axis=0).reshape(1, _A)
        vg_s[...] += jnp.sum(st32, axis=0).reshape(1, _A)
        ve_s[...] += jnp.sum(est32, axis=0).reshape(1, _NBR)

        @pl.when(i == _GRID - 1)
        def _():
            w1 = w1_ref[...]
            w2 = w2_ref[...]
            w3 = w3_ref[...]
            bb = bb_ref[...]
            n = f32(_E)
            xz = xz_s[...]
            zz = zz_s[...]
            c11 = xz[:, :_A] * f32(_M)
            c12 = xz[:, _A:2 * _A]
            c13 = xz[:, 2 * _A:]
            c22 = zz[:_A, :_A]
            c23 = zz[:_A, _A:]
            q = zz[_A:, _A:]
            t1 = jnp.sum(w1 * jnp.dot(c11, w1, preferred_element_type=f32), axis=0)
            t2 = jnp.sum(w2 * jnp.dot(c22, w2, preferred_element_type=f32), axis=0)
            t3 = jnp.sum(w3 * jnp.dot(q, w3, preferred_element_type=f32), axis=0)
            c1 = jnp.sum(w1 * jnp.dot(c12, w2, preferred_element_type=f32), axis=0)
            c2 = jnp.sum(w1 * jnp.dot(c13, w3, preferred_element_type=f32), axis=0)
            c3 = jnp.sum(w2 * jnp.dot(c23, w3, preferred_element_type=f32), axis=0)
            d = (t1 + t2 + t3 + 2.0 * (c1 + c2 + c3)).reshape(1, 2 * _A)
            sdot = (
                f32(_M) * jnp.dot(vx_s[...], w1, preferred_element_type=f32)
                + jnp.dot(vg_s[...], w2, preferred_element_type=f32)
                + jnp.dot(ve_s[...], w3, preferred_element_type=f32)
                + n * bb
            )
            mu = sdot / n
            sumsq = d + 2.0 * bb * sdot - n * bb * bb
            var = sumsq / n - mu * mu
            sc = g1_ref[...] / jnp.sqrt(var + 1e-5)
            sc_ref[...] = sc
            bf_ref[...] = (bb - mu) * sc + b1_ref[...]

    return pl.pallas_call(
        body,
        grid=(_GRID,),
        in_specs=[
            pl.BlockSpec((_T, _A), lambda i: (i, 0)),
            pl.BlockSpec((_T, _M, _A), lambda i: (i, 0, 0)),
            pl.BlockSpec((_T, _M, _NBR), lambda i: (i, 0, 0)),
            pl.BlockSpec((_A, 2 * _A), lambda i: (0, 0)),
            pl.BlockSpec((_A, 2 * _A), lambda i: (0, 0)),
            pl.BlockSpec((_NBR, 2 * _A), lambda i: (0, 0)),
            pl.BlockSpec((1, 2 * _A), lambda i: (0, 0)),
            pl.BlockSpec((1, 2 * _A), lambda i: (0, 0)),
            pl.BlockSpec((1, 2 * _A), lambda i: (0, 0)),
        ],
        out_specs=[
            pl.BlockSpec((1, 2 * _A), lambda i: (0, 0)),
            pl.BlockSpec((1, 2 * _A), lambda i: (0, 0)),
        ],
        out_shape=[
            jax.ShapeDtypeStruct((1, 2 * _A), jnp.float32),
            jax.ShapeDtypeStruct((1, 2 * _A), jnp.float32),
        ],
        scratch_shapes=[
            pltpu.VMEM((_A, 2 * _A + _NBR), jnp.float32),
            pltpu.VMEM((_A + _NBR, _A + _NBR), jnp.float32),
            pltpu.VMEM((1, _A), jnp.float32),
            pltpu.VMEM((1, _A), jnp.float32),
            pltpu.VMEM((1, _NBR), jnp.float32),
        ],
    )(x_bf, gath3, nbr_bf, w[:_A], w[_A:2 * _A], w[2 * _A:],
      b.reshape(1, 2 * _A), g1v.reshape(1, 2 * _A), b1v.reshape(1, 2 * _A))


def _conv_pass2(x, gath3, nbr_fea, wf, bf):
    def body(x_ref, g_ref, e_ref, w1_ref, w23_ref, b_ref, o_ref, st_ref):
        gated = _gated_tile(x_ref, g_ref, e_ref, w1_ref, w23_ref, b_ref)
        filt = jax.nn.sigmoid(gated[:, :_A])
        core = jnp.log(1.0 + jnp.exp(gated[:, _A:]))
        prod = (filt * core).reshape(_T, _M, _A)
        ns = jnp.sum(prod, axis=1)
        o_ref[...] = ns
        s = jnp.sum(ns, axis=0).reshape(1, _A)
        ss = jnp.sum(ns * ns, axis=0).reshape(1, _A)
        part = jnp.concatenate([s, ss, jnp.zeros((6, _A), jnp.float32)], axis=0)

        @pl.when(pl.program_id(0) == 0)
        def _():
            st_ref[...] = jnp.zeros_like(st_ref)

        st_ref[...] += part

    return pl.pallas_call(
        body,
        grid=(_GRID,),
        in_specs=[
            pl.BlockSpec((_T, _A), lambda i: (i, 0)),
            pl.BlockSpec((_T, _M, _A), lambda i: (i, 0, 0)),
            pl.BlockSpec((_T, _M, _NBR), lambda i: (i, 0, 0)),
            pl.BlockSpec((_A, 2 * _A), lambda i: (0, 0)),
            pl.BlockSpec((_A + _NBR, 2 * _A), lambda i: (0, 0)),
            pl.BlockSpec((1, 2 * _A), lambda i: (0, 0)),
        ],
        out_specs=[
            pl.BlockSpec((_T, _A), lambda i: (i, 0)),
            pl.BlockSpec((8, _A), lambda i: (0, 0)),
        ],
        out_shape=[
            jax.ShapeDtypeStruct((_N, _A), jnp.float32),
            jax.ShapeDtypeStruct((8, _A), jnp.float32),
        ],
    )(x, gath3, nbr_fea, wf[:_A], wf[_A:], bf)


def _bn_resid(x, ns, scale, shift):
    tm = 2000

    def body(x_ref, n_ref, sc_ref, sh_ref, o_ref):
        o_ref[...] = _softplus(x_ref[...] + n_ref[...] * sc_ref[...] + sh_ref[...])

    return pl.pallas_call(
        body,
        grid=(_N // tm,),
        in_specs=[
            pl.BlockSpec((tm, _A), lambda i: (i, 0)),
            pl.BlockSpec((tm, _A), lambda i: (i, 0)),
            pl.BlockSpec((1, _A), lambda i: (0, 0)),
            pl.BlockSpec((1, _A), lambda i: (0, 0)),
        ],
        out_specs=pl.BlockSpec((tm, _A), lambda i: (i, 0)),
        out_shape=jax.ShapeDtypeStruct((_N, _A), jnp.float32),
    )(x, ns, scale, shift)


def _pool_head(x3, t2, cw, cb, f0w, f0b, f1w, f1b, ow, ob):
    apc = _N // _B

    def body(x_ref, t_ref, cw_ref, cb_ref, f0w_ref, f0b_ref, f1w_ref, f1b_ref,
             ow_ref, ob_ref, o_ref):
        xv = x_ref[...]
        tv = t_ref[...]
        pools = []
        for eid in (_NI, _CU):
            mask = (tv == eid).astype(jnp.float32)
            cnt = jnp.sum(mask, axis=1)
            ssum = jnp.sum(xv * mask[:, :, None], axis=1)
            pooled = jnp.where(
                cnt[:, None] > 0, ssum / jnp.maximum(cnt, 1.0)[:, None], 0.0
            )
            pools.append(pooled)
        crys = _softplus(jnp.concatenate(pools, axis=1))
        crys = _softplus(
            jnp.dot(crys, cw_ref[...], preferred_element_type=jnp.float32)
            + cb_ref[...]
        )
        crys = _softplus(
            jnp.dot(crys, f0w_ref[...], preferred_element_type=jnp.float32)
            + f0b_ref[...]
        )
        crys = _softplus(
            jnp.dot(crys, f1w_ref[...], preferred_element_type=jnp.float32)
            + f1b_ref[...]
        )
        o_ref[...] = (
            jnp.dot(crys, ow_ref[...], preferred_element_type=jnp.float32)
            + ob_ref[...]
        )

    return pl.pallas_call(
        body,
        grid=(1,),
        in_specs=[
            pl.BlockSpec((_B, apc, _A), lambda i: (0, 0, 0)),
            pl.BlockSpec((_B, apc), lambda i: (0, 0)),
            pl.BlockSpec((2 * _A, _H), lambda i: (0, 0)),
            pl.BlockSpec((1, _H), lambda i: (0, 0)),
            pl.BlockSpec((_H, _H), lambda i: (0, 0)),
            pl.BlockSpec((1, _H), lambda i: (0, 0)),
            pl.BlockSpec((_H, _H), lambda i: (0, 0)),
            pl.BlockSpec((1, _H), lambda i: (0, 0)),
            pl.BlockSpec((_H, 1), lambda i: (0, 0)),
            pl.BlockSpec((1, 1), lambda i: (0, 0)),
        ],
        out_specs=pl.BlockSpec((_B, 1), lambda i: (0, 0)),
        out_shape=jax.ShapeDtypeStruct((_B, 1), jnp.float32),
    )(x3, t2, cw, cb, f0w, f0b, f1w, f1b, ow, ob)


def kernel(atom_fea, nbr_fea, nbr_fea_idx, crystal_atom_idx, atom_types,
           emb_W, emb_b, fc_W, fc_b, bn1_g, bn1_b, bn2_g, bn2_b,
           ctf_W, ctf_b, fcs_W, fcs_b, out_W, out_b):
    idx_flat = nbr_fea_idx.reshape(_E).astype(jnp.int32)
    nbr_bf = nbr_fea.astype(jnp.bfloat16)
    x = _embed(atom_fea, emb_W, emb_b)
    n1 = jnp.float32(_E)
    n2 = jnp.float32(_N)
    for i in range(_NCONV):
        x_bf = x.astype(jnp.bfloat16)
        gath3 = _sc_gather(idx_flat, x).reshape(_N, _M, _A)
        w, b = fc_W[i], fc_b[i]
        sc1, bfold = _gram_stats(x_bf, gath3, nbr_bf, w, b, bn1_g[i], bn1_b[i])
        wf = (w * sc1).astype(jnp.bfloat16)
        ns, st2 = _conv_pass2(x_bf, gath3, nbr_bf, wf, bfold)
        mu2 = st2[0] / n2
        var2 = st2[1] / n2 - mu2 * mu2
        sc2 = bn2_g[i] / jnp.sqrt(var2 + 1e-5)
        sh2 = bn2_b[i] - mu2 * sc2
        x = _bn_resid(x, ns, sc2.reshape(1, _A), sh2.reshape(1, _A))
    x3 = x.reshape(_B, _N // _B, _A)
    t2 = atom_types.reshape(_B, _N // _B).astype(jnp.int32)
    return _pool_head(
        x3, t2, ctf_W, ctf_b.reshape(1, _H),
        fcs_W[0], fcs_b[0].reshape(1, _H), fcs_W[1], fcs_b[1].reshape(1, _H),
        out_W, out_b.reshape(1, 1),
    )


# speedup vs baseline: 1.5315x; 1.0033x over previous
import functools

import jax
import jax.numpy as jnp
from jax import lax
from jax.experimental import pallas as pl
from jax.experimental.pallas import tpu as pltpu
from jax.experimental.pallas import tpu_sc as plsc

_N = 10000
_M = 32
_A = 128
_NBR = 16
_NCONV = 3
_H = 192
_B = 100
_NI = 28
_CU = 29
_E = _N * _M
_NW = 32
_PW = _E // _NW
_CH = 80
_NCH = _PW // _CH

_T = 400
_GRID = _N // _T


def _softplus(x):
    return jnp.maximum(x, 0.0) + jnp.log(1.0 + jnp.exp(-jnp.abs(x)))


def _sigmoid(x):
    return 1.0 / (1.0 + jnp.exp(-x))


def _sc_gather(idx_flat, table):
    mesh = plsc.VectorSubcoreMesh(core_axis_name="c", subcore_axis_name="s")

    @functools.partial(
        pl.kernel,
        out_type=jax.ShapeDtypeStruct((_E, _A), jnp.float32),
        mesh=mesh,
        scratch_types=[
            pltpu.VMEM((_PW,), jnp.int32),
            pltpu.VMEM((_CH, _A), jnp.float32),
            pltpu.VMEM((_CH, _A), jnp.float32),
            pltpu.VMEM_SHARED((_N, _A), jnp.float32),
            pltpu.SemaphoreType.DMA,
            pltpu.SemaphoreType.DMA,
        ],
    )
    def gk(idx_hbm, tab_hbm, out_hbm, idx_v, rows_a, rows_b, tab_s, g0, g1):
        wid = lax.axis_index("s") * 2 + lax.axis_index("c")
        sid = lax.axis_index("s")
        base = wid * _PW

        @pl.when(sid < 10)
        def _():
            pltpu.sync_copy(
                tab_hbm.at[pl.ds(sid * 1000, 1000)],
                tab_s.at[pl.ds(sid * 1000, 1000)],
            )
        pltpu.sync_copy(idx_hbm.at[pl.ds(base, _PW)], idx_v)
        plsc.subcore_barrier()
        gsem = (g0, g1)
        bufs = (rows_a, rows_b)

        def gather(c, b):
            pltpu.async_copy(
                tab_s.at[idx_v.at[pl.ds(c * _CH, _CH)]], bufs[b], gsem[b]
            )

        def gather_wait(c, b):
            pltpu.make_async_copy(
                tab_s.at[idx_v.at[pl.ds(c * _CH, _CH)]], bufs[b], gsem[b]
            ).wait()

        gather(0, 0)

        def outer(o, carry):
            c0 = o * 2
            for b in (0, 1):
                c = c0 + b

                @pl.when(c + 1 < _NCH)
                def _():
                    gather(c + 1, 1 - b)

                @pl.when(c < _NCH)
                def _():
                    gather_wait(c, b)
                    pltpu.sync_copy(
                        bufs[b], out_hbm.at[pl.ds(base + c * _CH, _CH)]
                    )

            return carry

        lax.fori_loop(0, (_NCH + 1) // 2, outer, 0)

    return gk(idx_flat, table)


def _embed(atom_fea, emb_W, emb_b):
    tm = 2000

    def body(x_ref, w_ref, b_ref, o_ref):
        o_ref[...] = (
            jnp.dot(x_ref[...], w_ref[...], preferred_element_type=jnp.float32)
            + b_ref[...]
        )

    return pl.pallas_call(
        body,
        grid=(_N // tm,),
        in_specs=[
            pl.BlockSpec((tm, _A), lambda i: (i, 0)),
            pl.BlockSpec((_A, _A), lambda i: (0, 0)),
            pl.BlockSpec((1, _A), lambda i: (0, 0)),
        ],
        out_specs=pl.BlockSpec((tm, _A), lambda i: (i, 0)),
        out_shape=jax.ShapeDtypeStruct((_N, _A), jnp.float32),
    )(atom_fea, emb_W, emb_b.reshape(1, _A))


def _gated_tile(x_ref, g_ref, e_ref, w1_ref, w23_ref, b_ref):
    xw = jnp.dot(x_ref[...], w1_ref[...], preferred_element_type=jnp.float32)
    g2 = g_ref[...].reshape(_T * _M, _A).astype(jnp.bfloat16)
    e2 = e_ref[...].reshape(_T * _M, _NBR)
    ge = jnp.concatenate([g2, e2], axis=1)
    gew = jnp.dot(ge, w23_ref[...], preferred_element_type=jnp.float32)
    gated3 = gew.reshape(_T, _M, 2 * _A) + xw[:, None, :] + b_ref[...]
    return gated3.reshape(_T * _M, 2 * _A)


def _gram_stats(x_bf, gath3, nbr_bf, w, b, g1v, b1v):
    f32 = jnp.float32
    dn = (((0,), (0,)), ((), ()))

    def body(x_ref, g_ref, e_ref, w1_ref, w2_ref, w3_ref, bb_ref, g1_ref, b1_ref,
             sc_ref, bf_ref, xz_s, zz_s, vx_s, vg_s, ve_s):
        i = pl.program_id(0)
        xt = x_ref[...]
        g32 = g_ref[...]
        gt = g32.astype(jnp.bfloat16).reshape(_T * _M, _A)
        et = e_ref[...].reshape(_T * _M, _NBR)
        st32 = jnp.sum(g32, axis=1)
        est32 = jnp.sum(e_ref[...].astype(f32), axis=1)
        stb = st32.astype(jnp.bfloat16)
        estb = est32.astype(jnp.bfloat16)

        @pl.when(i == 0)
        def _():
            xz_s[...] = jnp.zeros_like(xz_s)
            zz_s[...] = jnp.zeros_like(zz_s)
            vx_s[...] = jnp.zeros_like(vx_s)
            vg_s[...] = jnp.zeros_like(vg_s)
            ve_s[...] = jnp.zeros_like(ve_s)

        ze = jnp.concatenate([gt, et], axis=1)
        xs = jnp.concatenate([xt, stb, estb], axis=1)
        xz_s[...] += lax.dot_general(xt, xs, dn, preferred_element_type=f32)
        zz_s[...] += lax.dot_general(ze, ze, dn, preferred_element_type=f32)
        vx_s[...] += jnp.sum(xt.astype(f32), axis=0).reshape(1, _A)
        vg_s[...] += jnp.sum(st32, axis=0).reshape(1, _A)
        ve_s[...] += jnp.sum(est32, axis=0).reshape(1, _NBR)

        @pl.when(i == _GRID - 1)
        def _():
            w1 = w1_ref[...]
            w2 = w2_ref[...]
            w3 = w3_ref[...]
            bb = bb_ref[...]
            n = f32(_E)
            xz = xz_s[...]
            zz = zz_s[...]
            c11 = xz[:, :_A] * f32(_M)
            c12 = xz[:, _A:2 * _A]
            c13 = xz[:, 2 * _A:]
            c22 = zz[:_A, :_A]
            c23 = zz[:_A, _A:]
            q = zz[_A:, _A:]
            t1 = jnp.sum(w1 * jnp.dot(c11, w1, preferred_element_type=f32), axis=0)
            t2 = jnp.sum(w2 * jnp.dot(c22, w2, preferred_element_type=f32), axis=0)
            t3 = jnp.sum(w3 * jnp.dot(q, w3, preferred_element_type=f32), axis=0)
            c1 = jnp.sum(w1 * jnp.dot(c12, w2, preferred_element_type=f32), axis=0)
            c2 = jnp.sum(w1 * jnp.dot(c13, w3, preferred_element_type=f32), axis=0)
            c3 = jnp.sum(w2 * jnp.dot(c23, w3, preferred_element_type=f32), axis=0)
            d = (t1 + t2 + t3 + 2.0 * (c1 + c2 + c3)).reshape(1, 2 * _A)
            sdot = (
                f32(_M) * jnp.dot(vx_s[...], w1, preferred_element_type=f32)
                + jnp.dot(vg_s[...], w2, preferred_element_type=f32)
                + jnp.dot(ve_s[...], w3, preferred_element_type=f32)
                + n * bb
            )
            mu = sdot / n
            sumsq = d + 2.0 * bb * sdot - n * bb * bb
            var = sumsq / n - mu * mu
            sc = g1_ref[...] / jnp.sqrt(var + 1e-5)
            sc_ref[...] = sc
            bf_ref[...] = (bb - mu) * sc + b1_ref[...]

    return pl.pallas_call(
        body,
        grid=(_GRID,),
        in_specs=[
            pl.BlockSpec((_T, _A), lambda i: (i, 0)),
            pl.BlockSpec((_T, _M, _A), lambda i: (i, 0, 0)),
            pl.BlockSpec((_T, _M, _NBR), lambda i: (i, 0, 0)),
            pl.BlockSpec((_A, 2 * _A), lambda i: (0, 0)),
            pl.BlockSpec((_A, 2 * _A), lambda i: (0, 0)),
            pl.BlockSpec((_NBR, 2 * _A), lambda i: (0, 0)),
            pl.BlockSpec((1, 2 * _A), lambda i: (0, 0)),
            pl.BlockSpec((1, 2 * _A), lambda i: (0, 0)),
            pl.BlockSpec((1, 2 * _A), lambda i: (0, 0)),
        ],
        out_specs=[
            pl.BlockSpec((1, 2 * _A), lambda i: (0, 0)),
            pl.BlockSpec((1, 2 * _A), lambda i: (0, 0)),
        ],
        out_shape=[
            jax.ShapeDtypeStruct((1, 2 * _A), jnp.float32),
            jax.ShapeDtypeStruct((1, 2 * _A), jnp.float32),
        ],
        scratch_shapes=[
            pltpu.VMEM((_A, 2 * _A + _NBR), jnp.float32),
            pltpu.VMEM((_A + _NBR, _A + _NBR), jnp.float32),
            pltpu.VMEM((1, _A), jnp.float32),
            pltpu.VMEM((1, _A), jnp.float32),
            pltpu.VMEM((1, _NBR), jnp.float32),
        ],
    )(x_bf, gath3, nbr_bf, w[:_A], w[_A:2 * _A], w[2 * _A:],
      b.reshape(1, 2 * _A), g1v.reshape(1, 2 * _A), b1v.reshape(1, 2 * _A))


def _conv_pass2(x, gath3, nbr_fea, wf, bf):
    def body(x_ref, g_ref, e_ref, w1_ref, w23_ref, b_ref, o_ref, st_ref):
        gated = _gated_tile(x_ref, g_ref, e_ref, w1_ref, w23_ref, b_ref)
        filt = jax.nn.sigmoid(gated[:, :_A])
        core = jnp.log(1.0 + jnp.exp(gated[:, _A:]))
        prod = (filt * core).reshape(_T, _M, _A)
        ns = jnp.sum(prod, axis=1)
        o_ref[...] = ns
        s = jnp.sum(ns, axis=0).reshape(1, _A)
        ss = jnp.sum(ns * ns, axis=0).reshape(1, _A)
        part = jnp.concatenate([s, ss, jnp.zeros((6, _A), jnp.float32)], axis=0)

        @pl.when(pl.program_id(0) == 0)
        def _():
            st_ref[...] = jnp.zeros_like(st_ref)

        st_ref[...] += part

    return pl.pallas_call(
        body,
        grid=(_GRID,),
        in_specs=[
            pl.BlockSpec((_T, _A), lambda i: (i, 0)),
            pl.BlockSpec((_T, _M, _A), lambda i: (i, 0, 0)),
            pl.BlockSpec((_T, _M, _NBR), lambda i: (i, 0, 0)),
            pl.BlockSpec((_A, 2 * _A), lambda i: (0, 0)),
            pl.BlockSpec((_A + _NBR, 2 * _A), lambda i: (0, 0)),
            pl.BlockSpec((1, 2 * _A), lambda i: (0, 0)),
        ],
        out_specs=[
            pl.BlockSpec((_T, _A), lambda i: (i, 0)),
            pl.BlockSpec((8, _A), lambda i: (0, 0)),
        ],
        out_shape=[
            jax.ShapeDtypeStruct((_N, _A), jnp.float32),
            jax.ShapeDtypeStruct((8, _A), jnp.float32),
        ],
    )(x, gath3, nbr_fea, wf[:_A], wf[_A:], bf)


def _bn_resid(x, ns, scale, shift):
    tm = 2000

    def body(x_ref, n_ref, sc_ref, sh_ref, o_ref):
        o_ref[...] = _softplus(x_ref[...] + n_ref[...] * sc_ref[...] + sh_ref[...])

    return pl.pallas_call(
        body,
        grid=(_N // tm,),
        in_specs=[
            pl.BlockSpec((tm, _A), lambda i: (i, 0)),
            pl.BlockSpec((tm, _A), lambda i: (i, 0)),
            pl.BlockSpec((1, _A), lambda i: (0, 0)),
            pl.BlockSpec((1, _A), lambda i: (0, 0)),
        ],
        out_specs=pl.BlockSpec((tm, _A), lambda i: (i, 0)),
        out_shape=jax.ShapeDtypeStruct((_N, _A), jnp.float32),
    )(x, ns, scale, shift)


def _pool_head(x3, t2, cw, cb, f0w, f0b, f1w, f1b, ow, ob):
    apc = _N // _B

    def body(x_ref, t_ref, cw_ref, cb_ref, f0w_ref, f0b_ref, f1w_ref, f1b_ref,
             ow_ref, ob_ref, o_ref):
        xv = x_ref[...]
        tv = t_ref[...]
        pools = []
        for eid in (_NI, _CU):
            mask = (tv == eid).astype(jnp.float32)
            cnt = jnp.sum(mask, axis=1)
            ssum = jnp.sum(xv * mask[:, :, None], axis=1)
            pooled = jnp.where(
                cnt[:, None] > 0, ssum / jnp.maximum(cnt, 1.0)[:, None], 0.0
            )
            pools.append(pooled)
        crys = _softplus(jnp.concatenate(pools, axis=1))
        crys = _softplus(
            jnp.dot(crys, cw_ref[...], preferred_element_type=jnp.float32)
            + cb_ref[...]
        )
        crys = _softplus(
            jnp.dot(crys, f0w_ref[...], preferred_element_type=jnp.float32)
            + f0b_ref[...]
        )
        crys = _softplus(
            jnp.dot(crys, f1w_ref[...], preferred_element_type=jnp.float32)
            + f1b_ref[...]
        )
        o_ref[...] = (
            jnp.dot(crys, ow_ref[...], preferred_element_type=jnp.float32)
            + ob_ref[...]
        )

    return pl.pallas_call(
        body,
        grid=(1,),
        in_specs=[
            pl.BlockSpec((_B, apc, _A), lambda i: (0, 0, 0)),
            pl.BlockSpec((_B, apc), lambda i: (0, 0)),
            pl.BlockSpec((2 * _A, _H), lambda i: (0, 0)),
            pl.BlockSpec((1, _H), lambda i: (0, 0)),
            pl.BlockSpec((_H, _H), lambda i: (0, 0)),
            pl.BlockSpec((1, _H), lambda i: (0, 0)),
            pl.BlockSpec((_H, _H), lambda i: (0, 0)),
            pl.BlockSpec((1, _H), lambda i: (0, 0)),
            pl.BlockSpec((_H, 1), lambda i: (0, 0)),
            pl.BlockSpec((1, 1), lambda i: (0, 0)),
        ],
        out_specs=pl.BlockSpec((_B, 1), lambda i: (0, 0)),
        out_shape=jax.ShapeDtypeStruct((_B, 1), jnp.float32),
    )(x3, t2, cw, cb, f0w, f0b, f1w, f1b, ow, ob)


def kernel(atom_fea, nbr_fea, nbr_fea_idx, crystal_atom_idx, atom_types,
           emb_W, emb_b, fc_W, fc_b, bn1_g, bn1_b, bn2_g, bn2_b,
           ctf_W, ctf_b, fcs_W, fcs_b, out_W, out_b):
    idx_flat = nbr_fea_idx.reshape(_E).astype(jnp.int32)
    nbr_bf = nbr_fea.astype(jnp.bfloat16)
    x = _embed(atom_fea, emb_W, emb_b)
    n1 = jnp.float32(_E)
    n2 = jnp.float32(_N)
    for i in range(_NCONV):
        x_bf = x.astype(jnp.bfloat16)
        gath3 = _sc_gather(idx_flat, x).reshape(_N, _M, _A)
        w, b = fc_W[i], fc_b[i]
        sc1, bfold = _gram_stats(x_bf, gath3, nbr_bf, w, b, bn1_g[i], bn1_b[i])
        wf = (w * sc1).astype(jnp.bfloat16)
        ns, st2 = _conv_pass2(x_bf, gath3, nbr_bf, wf, bfold)
        mu2 = st2[0] / n2
        var2 = st2[1] / n2 - mu2 * mu2
        sc2 = bn2_g[i] / jnp.sqrt(var2 + 1e-5)
        sh2 = bn2_b[i] - mu2 * sc2
        x = _bn_resid(x, ns, sc2.reshape(1, _A), sh2.reshape(1, _A))
    x3 = x.reshape(_B, _N // _B, _A)
    t2 = atom_types.reshape(_B, _N // _B).astype(jnp.int32)
    return _pool_head(
        x3, t2, ctf_W, ctf_b.reshape(1, _H),
        fcs_W[0], fcs_b[0].reshape(1, _H), fcs_W[1], fcs_b[1].reshape(1, _H),
        out_W, out_b.reshape(1, 1),
    )
